# Initial kernel scaffold; baseline (speedup 1.0000x reference)
import jax
import jax.numpy as jnp
from jax.experimental import pallas as pl


def kernel(x, table):
    raise NotImplementedError("write your pallas kernel here")


# speedup vs baseline: 3.6718x; 3.6718x over previous
import jax
import jax.numpy as jnp
from jax.experimental import pallas as pl

ROW_BLOCK = 1024


def _copy_kernel(table_ref, out_ref):
    out_ref[...] = table_ref[...]


def kernel(x, table):
    seq_len = x.shape[1]
    embed_dim = table.shape[1]
    assert seq_len % ROW_BLOCK == 0
    grid = (seq_len // ROW_BLOCK,)
    return pl.pallas_call(
        _copy_kernel,
        grid=grid,
        in_specs=[pl.BlockSpec((ROW_BLOCK, embed_dim), lambda i: (i, 0))],
        out_specs=pl.BlockSpec((ROW_BLOCK, embed_dim), lambda i: (i, 0)),
        out_shape=jax.ShapeDtypeStruct((seq_len, embed_dim), table.dtype),
    )(table)
